# baseline (speedup 1.0000x reference)
import functools
import math

import jax
import jax.numpy as jnp
from jax import lax
from jax.experimental import pallas as pl
from jax.experimental.pallas import tpu as pltpu
from jax.experimental.pallas import tpu_sc as plsc

_B = 4096
_R = 64
_NFF = 1024
_SCALE = math.sqrt(2.0 / _NFF)

_NC = 2
_NS = 16
_NW = _NC * _NS
_BPW = _B // _NW


_RP = 128


def _sc_gather_body(idx0_hbm, idx1_hbm, idx2_hbm, u0_hbm, u1_hbm, u2_hbm,
                    g0_hbm, g1_hbm, g2_hbm,
                    idx0_v, idx1_v, idx2_v, r0_v, r1_v, r2_v,
                    sem0, sem1, sem2):
    wid = lax.axis_index("s") * _NC + lax.axis_index("c")
    base = wid * _BPW
    pltpu.sync_copy(idx0_hbm.at[pl.ds(base, _BPW)], idx0_v)
    pltpu.sync_copy(idx1_hbm.at[pl.ds(base, _BPW)], idx1_v)
    pltpu.sync_copy(idx2_hbm.at[pl.ds(base, _BPW)], idx2_v)
    c0 = pltpu.async_copy(u0_hbm.at[idx0_v], r0_v, sem0)
    c1 = pltpu.async_copy(u1_hbm.at[idx1_v], r1_v, sem1)
    c2 = pltpu.async_copy(u2_hbm.at[idx2_v], r2_v, sem2)
    c0.wait()
    pltpu.sync_copy(r0_v, g0_hbm.at[pl.ds(base, _BPW)])
    c1.wait()
    pltpu.sync_copy(r1_v, g1_hbm.at[pl.ds(base, _BPW)])
    c2.wait()
    pltpu.sync_copy(r2_v, g2_hbm.at[pl.ds(base, _BPW)])


@functools.cache
def _sc_gather():
    return pl.kernel(
        _sc_gather_body,
        out_type=(
            jax.ShapeDtypeStruct((_B, _RP), jnp.float32),
            jax.ShapeDtypeStruct((_B, _RP), jnp.float32),
            jax.ShapeDtypeStruct((_B, _RP), jnp.float32),
        ),
        mesh=plsc.VectorSubcoreMesh(core_axis_name="c", subcore_axis_name="s"),
        scratch_types=[
            pltpu.VMEM((_BPW,), jnp.int32),
            pltpu.VMEM((_BPW,), jnp.int32),
            pltpu.VMEM((_BPW,), jnp.int32),
            pltpu.VMEM((_BPW, _RP), jnp.float32),
            pltpu.VMEM((_BPW, _RP), jnp.float32),
            pltpu.VMEM((_BPW, _RP), jnp.float32),
            pltpu.SemaphoreType.DMA,
            pltpu.SemaphoreType.DMA,
            pltpu.SemaphoreType.DMA,
        ],
    )


_NV = 100000
_PBLK = 2000


def _pad_body(u0, u1, u2, o0, o1, o2):
    z = jnp.zeros((_PBLK, _RP - _R), jnp.float32)
    o0[...] = jnp.concatenate([u0[...], z], axis=1)
    o1[...] = jnp.concatenate([u1[...], z], axis=1)
    o2[...] = jnp.concatenate([u2[...], z], axis=1)


@functools.cache
def _pad_call():
    grid = _NV // _PBLK
    in_spec = pl.BlockSpec((_PBLK, _R), lambda i: (i, 0))
    out_spec = pl.BlockSpec((_PBLK, _RP), lambda i: (i, 0))
    return pl.pallas_call(
        _pad_body,
        grid=(grid,),
        in_specs=[in_spec, in_spec, in_spec],
        out_specs=[out_spec, out_spec, out_spec],
        out_shape=[jax.ShapeDtypeStruct((_NV, _RP), jnp.float32)] * 3,
    )


def _mlp_body(g0, g1, g2, t, w0, w1, w2, wt, bff, wout, bout, out):
    acc = jnp.dot(g0[...], w0[...], preferred_element_type=jnp.float32)
    acc += jnp.dot(g1[...], w1[...], preferred_element_type=jnp.float32)
    acc += jnp.dot(g2[...], w2[...], preferred_element_type=jnp.float32)
    acc += t[...] * wt[...]
    acc += bff[...]
    feat = jnp.cos(acc) * _SCALE
    out[...] = jnp.dot(feat, wout[...], preferred_element_type=jnp.float32) + bout[...]


def _mlp_call(blk):
    grid = _B // blk
    const = lambda shape: pl.BlockSpec(shape, lambda i: (0, 0))
    return pl.pallas_call(
        _mlp_body,
        grid=(grid,),
        in_specs=[
            pl.BlockSpec((blk, _RP), lambda i: (i, 0)),
            pl.BlockSpec((blk, _RP), lambda i: (i, 0)),
            pl.BlockSpec((blk, _RP), lambda i: (i, 0)),
            pl.BlockSpec((blk, 1), lambda i: (i, 0)),
            const((_RP, _NFF)),
            const((_RP, _NFF)),
            const((_RP, _NFF)),
            const((1, _NFF)),
            const((1, _NFF)),
            const((_NFF, 1)),
            const((1, 1)),
        ],
        out_specs=pl.BlockSpec((blk, 1), lambda i: (i, 0)),
        out_shape=jax.ShapeDtypeStruct((_B, 1), jnp.float32),
    )


@jax.jit
def kernel(b_i_n, b_t_n, U0, U1, U2, W_ff, b_ff, W_out, b_out):
    idx0 = b_i_n[:, 0]
    idx1 = b_i_n[:, 1]
    idx2 = b_i_n[:, 2]
    u0p, u1p, u2p = _pad_call()(U0, U1, U2)
    g0, g1, g2 = _sc_gather()(idx0, idx1, idx2, u0p, u1p, u2p)
    wpad = ((0, _RP - _R), (0, 0))
    w0 = jnp.pad(W_ff[0:_R], wpad)
    w1 = jnp.pad(W_ff[_R:2 * _R], wpad)
    w2 = jnp.pad(W_ff[2 * _R:3 * _R], wpad)
    wt = W_ff[3 * _R:3 * _R + 1]
    y = _mlp_call(512)(
        g0, g1, g2, b_t_n.reshape(_B, 1),
        w0, w1, w2, wt, b_ff.reshape(1, _NFF),
        W_out, b_out.reshape(1, 1),
    )
    return y


# speedup vs baseline: 1.1155x; 1.1155x over previous
import functools
import math

import jax
import jax.numpy as jnp
from jax import lax
from jax.experimental import pallas as pl
from jax.experimental.pallas import tpu as pltpu
from jax.experimental.pallas import tpu_sc as plsc

_B = 4096
_R = 64
_NFF = 1024
_SCALE = math.sqrt(2.0 / _NFF)

_NC = 2
_NS = 16
_NW = _NC * _NS
_BPW = _B // _NW


_RP = 128


def _sc_gather_body(idx0_hbm, idx1_hbm, idx2_hbm, u0_hbm, u1_hbm, u2_hbm,
                    g0_hbm, g1_hbm, g2_hbm,
                    idx0_v, idx1_v, idx2_v, r0_v, r1_v, r2_v,
                    sem0, sem1, sem2):
    wid = lax.axis_index("s") * _NC + lax.axis_index("c")
    base = wid * _BPW
    pltpu.sync_copy(idx0_hbm.at[pl.ds(base, _BPW)], idx0_v)
    pltpu.sync_copy(idx1_hbm.at[pl.ds(base, _BPW)], idx1_v)
    pltpu.sync_copy(idx2_hbm.at[pl.ds(base, _BPW)], idx2_v)
    c0 = pltpu.async_copy(u0_hbm.at[idx0_v], r0_v, sem0)
    c1 = pltpu.async_copy(u1_hbm.at[idx1_v], r1_v, sem1)
    c2 = pltpu.async_copy(u2_hbm.at[idx2_v], r2_v, sem2)
    c0.wait()
    pltpu.sync_copy(r0_v, g0_hbm.at[pl.ds(base, _BPW)])
    c1.wait()
    pltpu.sync_copy(r1_v, g1_hbm.at[pl.ds(base, _BPW)])
    c2.wait()
    pltpu.sync_copy(r2_v, g2_hbm.at[pl.ds(base, _BPW)])


@functools.cache
def _sc_gather():
    return pl.kernel(
        _sc_gather_body,
        out_type=(
            jax.ShapeDtypeStruct((_B, _RP), jnp.float32),
            jax.ShapeDtypeStruct((_B, _RP), jnp.float32),
            jax.ShapeDtypeStruct((_B, _RP), jnp.float32),
        ),
        mesh=plsc.VectorSubcoreMesh(core_axis_name="c", subcore_axis_name="s"),
        scratch_types=[
            pltpu.VMEM((_BPW,), jnp.int32),
            pltpu.VMEM((_BPW,), jnp.int32),
            pltpu.VMEM((_BPW,), jnp.int32),
            pltpu.VMEM((_BPW, _RP), jnp.float32),
            pltpu.VMEM((_BPW, _RP), jnp.float32),
            pltpu.VMEM((_BPW, _RP), jnp.float32),
            pltpu.SemaphoreType.DMA,
            pltpu.SemaphoreType.DMA,
            pltpu.SemaphoreType.DMA,
        ],
    )


_NV = 100000
_PBLK = 2000


def _pad_body(u0, o0):
    z = jnp.zeros((_PBLK, _RP - _R), jnp.float32)
    o0[...] = jnp.concatenate([u0[...], z], axis=1)


@functools.cache
def _pad_call():
    grid = _NV // _PBLK
    return pl.pallas_call(
        _pad_body,
        grid=(grid,),
        in_specs=[pl.BlockSpec((_PBLK, _R), lambda i: (i, 0))],
        out_specs=pl.BlockSpec((_PBLK, _RP), lambda i: (i, 0)),
        out_shape=jax.ShapeDtypeStruct((_NV, _RP), jnp.float32),
    )


def _mlp_body(g0, g1, g2, t, w0, w1, w2, wt, bff, wout, bout, out):
    acc = jnp.dot(g0[...], w0[...], preferred_element_type=jnp.float32)
    acc += jnp.dot(g1[...], w1[...], preferred_element_type=jnp.float32)
    acc += jnp.dot(g2[...], w2[...], preferred_element_type=jnp.float32)
    acc += t[...] * wt[...]
    acc += bff[...]
    feat = jnp.cos(acc) * _SCALE
    out[...] = jnp.dot(feat, wout[...], preferred_element_type=jnp.float32) + bout[...]


def _mlp_call(blk):
    grid = _B // blk
    const = lambda shape: pl.BlockSpec(shape, lambda i: (0, 0))
    return pl.pallas_call(
        _mlp_body,
        grid=(grid,),
        in_specs=[
            pl.BlockSpec((blk, _RP), lambda i: (i, 0)),
            pl.BlockSpec((blk, _RP), lambda i: (i, 0)),
            pl.BlockSpec((blk, _RP), lambda i: (i, 0)),
            pl.BlockSpec((blk, 1), lambda i: (i, 0)),
            const((_RP, _NFF)),
            const((_RP, _NFF)),
            const((_RP, _NFF)),
            const((1, _NFF)),
            const((1, _NFF)),
            const((_NFF, 1)),
            const((1, 1)),
        ],
        out_specs=pl.BlockSpec((blk, 1), lambda i: (i, 0)),
        out_shape=jax.ShapeDtypeStruct((_B, 1), jnp.float32),
    )


@jax.jit
def kernel(b_i_n, b_t_n, U0, U1, U2, W_ff, b_ff, W_out, b_out):
    idx0 = b_i_n[:, 0]
    idx1 = b_i_n[:, 1]
    idx2 = b_i_n[:, 2]
    tpad = ((0, 0), (0, _RP - _R))
    u0p = jnp.pad(U0, tpad)
    u1p = jnp.pad(U1, tpad)
    u2p = _pad_call()(U2)
    g0, g1, g2 = _sc_gather()(idx0, idx1, idx2, u0p, u1p, u2p)
    wpad = ((0, _RP - _R), (0, 0))
    w0 = jnp.pad(W_ff[0:_R], wpad)
    w1 = jnp.pad(W_ff[_R:2 * _R], wpad)
    w2 = jnp.pad(W_ff[2 * _R:3 * _R], wpad)
    wt = W_ff[3 * _R:3 * _R + 1]
    y = _mlp_call(512)(
        g0, g1, g2, b_t_n.reshape(_B, 1),
        w0, w1, w2, wt, b_ff.reshape(1, _NFF),
        W_out, b_out.reshape(1, 1),
    )
    return y


# speedup vs baseline: 1.4173x; 1.2705x over previous
import functools
import math

import jax
import jax.numpy as jnp
from jax import lax
from jax.experimental import pallas as pl
from jax.experimental.pallas import tpu as pltpu
from jax.experimental.pallas import tpu_sc as plsc

_B = 4096
_R = 64
_NFF = 1024
_SCALE = math.sqrt(2.0 / _NFF)

_NC = 2
_NS = 16
_NW = _NC * _NS
_BPW = _B // _NW


_RP = 128


def _sc_gather_body(idx0_hbm, idx1_hbm, idx2_hbm, u0_hbm, u1_hbm, u2_hbm,
                    g0_hbm, g1_hbm, g2_hbm,
                    idx0_v, idx1_v, idx2_v, r0_v, r1_v, r2_v,
                    sem0, sem1, sem2):
    wid = lax.axis_index("s") * _NC + lax.axis_index("c")
    base = wid * _BPW
    pltpu.sync_copy(idx0_hbm.at[pl.ds(base, _BPW)], idx0_v)
    pltpu.sync_copy(idx1_hbm.at[pl.ds(base, _BPW)], idx1_v)
    pltpu.sync_copy(idx2_hbm.at[pl.ds(base, _BPW)], idx2_v)
    c0 = pltpu.async_copy(u0_hbm.at[idx0_v], r0_v, sem0)
    c1 = pltpu.async_copy(u1_hbm.at[idx1_v], r1_v, sem1)
    c2 = pltpu.async_copy(u2_hbm.at[idx2_v], r2_v, sem2)
    c0.wait()
    pltpu.sync_copy(r0_v, g0_hbm.at[pl.ds(base, _BPW)])
    c1.wait()
    pltpu.sync_copy(r1_v, g1_hbm.at[pl.ds(base, _BPW)])
    c2.wait()
    pltpu.sync_copy(r2_v, g2_hbm.at[pl.ds(base, _BPW)])


@functools.cache
def _sc_gather():
    return pl.kernel(
        _sc_gather_body,
        out_type=(
            jax.ShapeDtypeStruct((_B, _RP), jnp.float32),
            jax.ShapeDtypeStruct((_B, _RP), jnp.float32),
            jax.ShapeDtypeStruct((_B, _RP), jnp.float32),
        ),
        mesh=plsc.VectorSubcoreMesh(core_axis_name="c", subcore_axis_name="s"),
        scratch_types=[
            pltpu.VMEM((_BPW,), jnp.int32),
            pltpu.VMEM((_BPW,), jnp.int32),
            pltpu.VMEM((_BPW,), jnp.int32),
            pltpu.VMEM((_BPW, _RP), jnp.float32),
            pltpu.VMEM((_BPW, _RP), jnp.float32),
            pltpu.VMEM((_BPW, _RP), jnp.float32),
            pltpu.SemaphoreType.DMA,
            pltpu.SemaphoreType.DMA,
            pltpu.SemaphoreType.DMA,
        ],
    )


_NV = 100000
_PBLK = 2000


def _pad_body(u0, o0):
    z = jnp.zeros((_PBLK, _RP - _R), jnp.float32)
    o0[...] = jnp.concatenate([u0[...], z], axis=1)


@functools.cache
def _pad_call():
    grid = _NV // _PBLK
    return pl.pallas_call(
        _pad_body,
        grid=(grid,),
        in_specs=[pl.BlockSpec((_PBLK, _R), lambda i: (i, 0))],
        out_specs=pl.BlockSpec((_PBLK, _RP), lambda i: (i, 0)),
        out_shape=jax.ShapeDtypeStruct((_NV, _RP), jnp.float32),
    )


def _fast_cos(x):
    k = jnp.round(x * 0.6366197723675814)
    ki = k.astype(jnp.int32)
    r = x - k * 1.5707855224609375
    r = r - k * 1.0804334124e-5
    r = r - k * 6.0771e-11
    z = r * r
    cosp = ((2.443315711809948e-5 * z - 1.388731625493765e-3) * z
            + 4.166664568298827e-2) * z * z - 0.5 * z + 1.0
    sinp = (((-1.9515295891e-4 * z + 8.3321608736e-3) * z
             - 1.6666654611e-1) * z) * r + r
    m1 = ki & 1
    m2 = (ki >> 1) & 1
    res = jnp.where(m1 == 1, sinp, cosp)
    return jnp.where((m1 ^ m2) == 1, -res, res)


def _mlp_body(g0, g1, g2, t, w0, w1, w2, wt, bff, wout, bout, out):
    acc = jnp.dot(g0[...], w0[...], preferred_element_type=jnp.float32)
    acc += jnp.dot(g1[...], w1[...], preferred_element_type=jnp.float32)
    acc += jnp.dot(g2[...], w2[...], preferred_element_type=jnp.float32)
    acc += t[...] * wt[...]
    acc += bff[...]
    feat = _fast_cos(acc) * _SCALE
    out[...] = jnp.dot(feat, wout[...], preferred_element_type=jnp.float32) + bout[...]


def _mlp_call(blk):
    grid = _B // blk
    const = lambda shape: pl.BlockSpec(shape, lambda i: (0, 0))
    return pl.pallas_call(
        _mlp_body,
        grid=(grid,),
        in_specs=[
            pl.BlockSpec((blk, _RP), lambda i: (i, 0)),
            pl.BlockSpec((blk, _RP), lambda i: (i, 0)),
            pl.BlockSpec((blk, _RP), lambda i: (i, 0)),
            pl.BlockSpec((blk, 1), lambda i: (i, 0)),
            const((_RP, _NFF)),
            const((_RP, _NFF)),
            const((_RP, _NFF)),
            const((1, _NFF)),
            const((1, _NFF)),
            const((_NFF, 1)),
            const((1, 1)),
        ],
        out_specs=pl.BlockSpec((blk, 1), lambda i: (i, 0)),
        out_shape=jax.ShapeDtypeStruct((_B, 1), jnp.float32),
    )


@jax.jit
def kernel(b_i_n, b_t_n, U0, U1, U2, W_ff, b_ff, W_out, b_out):
    idx0 = b_i_n[:, 0]
    idx1 = b_i_n[:, 1]
    idx2 = b_i_n[:, 2]
    tpad = ((0, 0), (0, _RP - _R))
    u0p = jnp.pad(U0, tpad)
    u1p = jnp.pad(U1, tpad)
    u2p = jnp.pad(U2, tpad)
    g0, g1, g2 = _sc_gather()(idx0, idx1, idx2, u0p, u1p, u2p)
    wpad = ((0, _RP - _R), (0, 0))
    w0 = jnp.pad(W_ff[0:_R], wpad)
    w1 = jnp.pad(W_ff[_R:2 * _R], wpad)
    w2 = jnp.pad(W_ff[2 * _R:3 * _R], wpad)
    wt = W_ff[3 * _R:3 * _R + 1]
    y = _mlp_call(512)(
        g0, g1, g2, b_t_n.reshape(_B, 1),
        w0, w1, w2, wt, b_ff.reshape(1, _NFF),
        W_out, b_out.reshape(1, 1),
    )
    return y


# speedup vs baseline: 1.4281x; 1.0076x over previous
import functools
import math

import jax
import jax.numpy as jnp
from jax import lax
from jax.experimental import pallas as pl
from jax.experimental.pallas import tpu as pltpu
from jax.experimental.pallas import tpu_sc as plsc

_B = 4096
_R = 64
_NFF = 1024
_SCALE = math.sqrt(2.0 / _NFF)

_NC = 2
_NS = 16
_NW = _NC * _NS
_BPW = _B // _NW


_RP = 128


def _sc_gather_body(idx0_hbm, idx1_hbm, idx2_hbm, u0_hbm, u1_hbm, u2_hbm,
                    g0_hbm, g1_hbm, g2_hbm,
                    idx0_v, idx1_v, idx2_v, r0_v, r1_v, r2_v,
                    sem0, sem1, sem2):
    wid = lax.axis_index("s") * _NC + lax.axis_index("c")
    base = wid * _BPW
    pltpu.sync_copy(idx0_hbm.at[pl.ds(base, _BPW)], idx0_v)
    pltpu.sync_copy(idx1_hbm.at[pl.ds(base, _BPW)], idx1_v)
    pltpu.sync_copy(idx2_hbm.at[pl.ds(base, _BPW)], idx2_v)
    c0 = pltpu.async_copy(u0_hbm.at[idx0_v], r0_v, sem0)
    c1 = pltpu.async_copy(u1_hbm.at[idx1_v], r1_v, sem1)
    c2 = pltpu.async_copy(u2_hbm.at[idx2_v], r2_v, sem2)
    c0.wait()
    pltpu.sync_copy(r0_v, g0_hbm.at[pl.ds(base, _BPW)])
    c1.wait()
    pltpu.sync_copy(r1_v, g1_hbm.at[pl.ds(base, _BPW)])
    c2.wait()
    pltpu.sync_copy(r2_v, g2_hbm.at[pl.ds(base, _BPW)])


@functools.cache
def _sc_gather():
    return pl.kernel(
        _sc_gather_body,
        out_type=(
            jax.ShapeDtypeStruct((_B, _RP), jnp.float32),
            jax.ShapeDtypeStruct((_B, _RP), jnp.float32),
            jax.ShapeDtypeStruct((_B, _RP), jnp.float32),
        ),
        mesh=plsc.VectorSubcoreMesh(core_axis_name="c", subcore_axis_name="s"),
        scratch_types=[
            pltpu.VMEM((_BPW,), jnp.int32),
            pltpu.VMEM((_BPW,), jnp.int32),
            pltpu.VMEM((_BPW,), jnp.int32),
            pltpu.VMEM((_BPW, _RP), jnp.float32),
            pltpu.VMEM((_BPW, _RP), jnp.float32),
            pltpu.VMEM((_BPW, _RP), jnp.float32),
            pltpu.SemaphoreType.DMA,
            pltpu.SemaphoreType.DMA,
            pltpu.SemaphoreType.DMA,
        ],
    )


_NV = 100000
_RPW = 3120
_REMB = _NW * _RPW
_REM = _NV - _REMB


def _sc_repack_body(u0_hbm, u1_hbm, u2_hbm, o0_hbm, o1_hbm, o2_hbm,
                    sem0, sem1, sem2):
    wid = lax.axis_index("s") * _NC + lax.axis_index("c")
    base = wid * _RPW
    c0 = pltpu.async_copy(u0_hbm.at[pl.ds(base, _RPW)],
                          o0_hbm.at[pl.ds(base, _RPW), pl.ds(0, _R)], sem0)
    c1 = pltpu.async_copy(u1_hbm.at[pl.ds(base, _RPW)],
                          o1_hbm.at[pl.ds(base, _RPW), pl.ds(0, _R)], sem1)
    c2 = pltpu.async_copy(u2_hbm.at[pl.ds(base, _RPW)],
                          o2_hbm.at[pl.ds(base, _RPW), pl.ds(0, _R)], sem2)

    @pl.when(wid == 0)
    def _():
        pltpu.sync_copy(u0_hbm.at[pl.ds(_REMB, _REM)],
                        o0_hbm.at[pl.ds(_REMB, _REM), pl.ds(0, _R)])
        pltpu.sync_copy(u1_hbm.at[pl.ds(_REMB, _REM)],
                        o1_hbm.at[pl.ds(_REMB, _REM), pl.ds(0, _R)])
        pltpu.sync_copy(u2_hbm.at[pl.ds(_REMB, _REM)],
                        o2_hbm.at[pl.ds(_REMB, _REM), pl.ds(0, _R)])

    c0.wait()
    c1.wait()
    c2.wait()


@functools.cache
def _sc_repack():
    return pl.kernel(
        _sc_repack_body,
        out_type=(
            jax.ShapeDtypeStruct((_NV, _RP), jnp.float32),
            jax.ShapeDtypeStruct((_NV, _RP), jnp.float32),
            jax.ShapeDtypeStruct((_NV, _RP), jnp.float32),
        ),
        mesh=plsc.VectorSubcoreMesh(core_axis_name="c", subcore_axis_name="s"),
        scratch_types=[
            pltpu.SemaphoreType.DMA,
            pltpu.SemaphoreType.DMA,
            pltpu.SemaphoreType.DMA,
        ],
    )


def _fast_cos(x):
    k = jnp.round(x * 0.6366197723675814)
    ki = k.astype(jnp.int32)
    r = x - k * 1.5707855224609375
    r = r - k * 1.0804334124e-5
    r = r - k * 6.0771e-11
    z = r * r
    cosp = ((2.443315711809948e-5 * z - 1.388731625493765e-3) * z
            + 4.166664568298827e-2) * z * z - 0.5 * z + 1.0
    sinp = (((-1.9515295891e-4 * z + 8.3321608736e-3) * z
             - 1.6666654611e-1) * z) * r + r
    m1 = ki & 1
    m2 = (ki >> 1) & 1
    res = jnp.where(m1 == 1, sinp, cosp)
    return jnp.where((m1 ^ m2) == 1, -res, res)


def _mlp_body(g0, g1, g2, t, w0, w1, w2, wt, bff, wout, bout, out):
    acc = jnp.dot(g0[:, :_R], w0[...], preferred_element_type=jnp.float32)
    acc += jnp.dot(g1[:, :_R], w1[...], preferred_element_type=jnp.float32)
    acc += jnp.dot(g2[:, :_R], w2[...], preferred_element_type=jnp.float32)
    acc += t[...] * wt[...]
    acc += bff[...]
    feat = _fast_cos(acc) * _SCALE
    out[...] = jnp.dot(feat, wout[...], preferred_element_type=jnp.float32) + bout[...]


def _mlp_call(blk):
    grid = _B // blk
    const = lambda shape: pl.BlockSpec(shape, lambda i: (0, 0))
    return pl.pallas_call(
        _mlp_body,
        grid=(grid,),
        in_specs=[
            pl.BlockSpec((blk, _RP), lambda i: (i, 0)),
            pl.BlockSpec((blk, _RP), lambda i: (i, 0)),
            pl.BlockSpec((blk, _RP), lambda i: (i, 0)),
            pl.BlockSpec((blk, 1), lambda i: (i, 0)),
            const((_R, _NFF)),
            const((_R, _NFF)),
            const((_R, _NFF)),
            const((1, _NFF)),
            const((1, _NFF)),
            const((_NFF, 1)),
            const((1, 1)),
        ],
        out_specs=pl.BlockSpec((blk, 1), lambda i: (i, 0)),
        out_shape=jax.ShapeDtypeStruct((_B, 1), jnp.float32),
    )


@jax.jit
def kernel(b_i_n, b_t_n, U0, U1, U2, W_ff, b_ff, W_out, b_out):
    idx0 = b_i_n[:, 0]
    idx1 = b_i_n[:, 1]
    idx2 = b_i_n[:, 2]
    tpad = ((0, 0), (0, _RP - _R))
    u0p = jnp.pad(U0, tpad)
    u1p = jnp.pad(U1, tpad)
    u2p = jnp.pad(U2, tpad)
    g0, g1, g2 = _sc_gather()(idx0, idx1, idx2, u0p, u1p, u2p)
    w0 = W_ff[0:_R]
    w1 = W_ff[_R:2 * _R]
    w2 = W_ff[2 * _R:3 * _R]
    wt = W_ff[3 * _R:3 * _R + 1]
    y = _mlp_call(512)(
        g0, g1, g2, b_t_n.reshape(_B, 1),
        w0, w1, w2, wt, b_ff.reshape(1, _NFF),
        W_out, b_out.reshape(1, 1),
    )
    return y


# speedup vs baseline: 1.4334x; 1.0037x over previous
import functools
import math

import jax
import jax.numpy as jnp
from jax import lax
from jax.experimental import pallas as pl
from jax.experimental.pallas import tpu as pltpu
from jax.experimental.pallas import tpu_sc as plsc

_B = 4096
_R = 64
_NFF = 1024
_SCALE = math.sqrt(2.0 / _NFF)

_NC = 2
_NS = 16
_NW = _NC * _NS
_BPW = _B // _NW


_RP = 128


def _sc_gather_body(idx0_hbm, idx1_hbm, idx2_hbm, u0_hbm, u1_hbm, u2_hbm,
                    g0_hbm, g1_hbm, g2_hbm,
                    idx0_v, idx1_v, idx2_v, r0_v, r1_v, r2_v,
                    sem0, sem1, sem2):
    wid = lax.axis_index("s") * _NC + lax.axis_index("c")
    base = wid * _BPW
    pltpu.sync_copy(idx0_hbm.at[pl.ds(base, _BPW)], idx0_v)
    pltpu.sync_copy(idx1_hbm.at[pl.ds(base, _BPW)], idx1_v)
    pltpu.sync_copy(idx2_hbm.at[pl.ds(base, _BPW)], idx2_v)
    c0 = pltpu.async_copy(u0_hbm.at[idx0_v], r0_v, sem0)
    c1 = pltpu.async_copy(u1_hbm.at[idx1_v], r1_v, sem1)
    c2 = pltpu.async_copy(u2_hbm.at[idx2_v], r2_v, sem2)
    c0.wait()
    pltpu.sync_copy(r0_v, g0_hbm.at[pl.ds(base, _BPW)])
    c1.wait()
    pltpu.sync_copy(r1_v, g1_hbm.at[pl.ds(base, _BPW)])
    c2.wait()
    pltpu.sync_copy(r2_v, g2_hbm.at[pl.ds(base, _BPW)])


@functools.cache
def _sc_gather():
    return pl.kernel(
        _sc_gather_body,
        out_type=(
            jax.ShapeDtypeStruct((_B, _RP), jnp.float32),
            jax.ShapeDtypeStruct((_B, _RP), jnp.float32),
            jax.ShapeDtypeStruct((_B, _RP), jnp.float32),
        ),
        mesh=plsc.VectorSubcoreMesh(core_axis_name="c", subcore_axis_name="s"),
        scratch_types=[
            pltpu.VMEM((_BPW,), jnp.int32),
            pltpu.VMEM((_BPW,), jnp.int32),
            pltpu.VMEM((_BPW,), jnp.int32),
            pltpu.VMEM((_BPW, _RP), jnp.float32),
            pltpu.VMEM((_BPW, _RP), jnp.float32),
            pltpu.VMEM((_BPW, _RP), jnp.float32),
            pltpu.SemaphoreType.DMA,
            pltpu.SemaphoreType.DMA,
            pltpu.SemaphoreType.DMA,
        ],
    )


_NV = 100000
_RPW = 3120
_REMB = _NW * _RPW
_REM = _NV - _REMB


def _sc_repack_body(u0_hbm, u1_hbm, u2_hbm, o0_hbm, o1_hbm, o2_hbm,
                    sem0, sem1, sem2):
    wid = lax.axis_index("s") * _NC + lax.axis_index("c")
    base = wid * _RPW
    c0 = pltpu.async_copy(u0_hbm.at[pl.ds(base, _RPW)],
                          o0_hbm.at[pl.ds(base, _RPW), pl.ds(0, _R)], sem0)
    c1 = pltpu.async_copy(u1_hbm.at[pl.ds(base, _RPW)],
                          o1_hbm.at[pl.ds(base, _RPW), pl.ds(0, _R)], sem1)
    c2 = pltpu.async_copy(u2_hbm.at[pl.ds(base, _RPW)],
                          o2_hbm.at[pl.ds(base, _RPW), pl.ds(0, _R)], sem2)

    @pl.when(wid == 0)
    def _():
        pltpu.sync_copy(u0_hbm.at[pl.ds(_REMB, _REM)],
                        o0_hbm.at[pl.ds(_REMB, _REM), pl.ds(0, _R)])
        pltpu.sync_copy(u1_hbm.at[pl.ds(_REMB, _REM)],
                        o1_hbm.at[pl.ds(_REMB, _REM), pl.ds(0, _R)])
        pltpu.sync_copy(u2_hbm.at[pl.ds(_REMB, _REM)],
                        o2_hbm.at[pl.ds(_REMB, _REM), pl.ds(0, _R)])

    c0.wait()
    c1.wait()
    c2.wait()


@functools.cache
def _sc_repack():
    return pl.kernel(
        _sc_repack_body,
        out_type=(
            jax.ShapeDtypeStruct((_NV, _RP), jnp.float32),
            jax.ShapeDtypeStruct((_NV, _RP), jnp.float32),
            jax.ShapeDtypeStruct((_NV, _RP), jnp.float32),
        ),
        mesh=plsc.VectorSubcoreMesh(core_axis_name="c", subcore_axis_name="s"),
        scratch_types=[
            pltpu.SemaphoreType.DMA,
            pltpu.SemaphoreType.DMA,
            pltpu.SemaphoreType.DMA,
        ],
    )


def _fast_cos(x):
    k = jnp.round(x * 0.6366197723675814)
    ki = k.astype(jnp.int32)
    r = x - k * 1.5707855224609375
    r = r - k * 1.0804334124e-5
    r = r - k * 6.0771e-11
    z = r * r
    cosp = ((2.443315711809948e-5 * z - 1.388731625493765e-3) * z
            + 4.166664568298827e-2) * z * z - 0.5 * z + 1.0
    sinp = (((-1.9515295891e-4 * z + 8.3321608736e-3) * z
             - 1.6666654611e-1) * z) * r + r
    m1 = ki & 1
    m2 = (ki >> 1) & 1
    res = jnp.where(m1 == 1, sinp, cosp)
    return jnp.where((m1 ^ m2) == 1, -res, res)


def _mlp_body(g0, g1, g2, t, w0, w1, w2, wt, bff, wout, bout, out):
    acc = jnp.dot(g0[:, :_R], w0[...], preferred_element_type=jnp.float32)
    acc += jnp.dot(g1[:, :_R], w1[...], preferred_element_type=jnp.float32)
    acc += jnp.dot(g2[:, :_R], w2[...], preferred_element_type=jnp.float32)
    acc += t[...] * wt[...]
    acc += bff[...]
    feat = _fast_cos(acc) * _SCALE
    out[...] = jnp.dot(feat, wout[...], preferred_element_type=jnp.float32) + bout[...]


def _mlp_call(blk):
    grid = _B // blk
    const = lambda shape: pl.BlockSpec(shape, lambda i: (0, 0))
    return pl.pallas_call(
        _mlp_body,
        grid=(grid,),
        in_specs=[
            pl.BlockSpec((blk, _RP), lambda i: (i, 0)),
            pl.BlockSpec((blk, _RP), lambda i: (i, 0)),
            pl.BlockSpec((blk, _RP), lambda i: (i, 0)),
            pl.BlockSpec((blk, 1), lambda i: (i, 0)),
            const((_R, _NFF)),
            const((_R, _NFF)),
            const((_R, _NFF)),
            const((1, _NFF)),
            const((1, _NFF)),
            const((_NFF, 1)),
            const((1, 1)),
        ],
        out_specs=pl.BlockSpec((blk, 1), lambda i: (i, 0)),
        out_shape=jax.ShapeDtypeStruct((_B, 1), jnp.float32),
    )


@jax.jit
def kernel(b_i_n, b_t_n, U0, U1, U2, W_ff, b_ff, W_out, b_out):
    idx0 = b_i_n[:, 0]
    idx1 = b_i_n[:, 1]
    idx2 = b_i_n[:, 2]
    tpad = ((0, 0), (0, _RP - _R))
    u0p = jnp.pad(U0, tpad)
    u1p = jnp.pad(U1, tpad)
    u2p = jnp.pad(U2, tpad)
    g0, g1, g2 = _sc_gather()(idx0, idx1, idx2, u0p, u1p, u2p)
    w0 = W_ff[0:_R]
    w1 = W_ff[_R:2 * _R]
    w2 = W_ff[2 * _R:3 * _R]
    wt = W_ff[3 * _R:3 * _R + 1]
    y = _mlp_call(1024)(
        g0, g1, g2, b_t_n.reshape(_B, 1),
        w0, w1, w2, wt, b_ff.reshape(1, _NFF),
        W_out, b_out.reshape(1, 1),
    )
    return y


# speedup vs baseline: 1.4339x; 1.0004x over previous
import functools
import math

import jax
import jax.numpy as jnp
from jax import lax
from jax.experimental import pallas as pl
from jax.experimental.pallas import tpu as pltpu
from jax.experimental.pallas import tpu_sc as plsc

_B = 4096
_R = 64
_NFF = 1024
_SCALE = math.sqrt(2.0 / _NFF)

_NC = 2
_NS = 16
_NW = _NC * _NS
_BPW = _B // _NW


_RP = 128


def _sc_gather_body(idx0_hbm, idx1_hbm, idx2_hbm, u0_hbm, u1_hbm, u2_hbm,
                    g0_hbm, g1_hbm, g2_hbm,
                    idx0_v, idx1_v, idx2_v, r0_v, r1_v, r2_v,
                    sem0, sem1, sem2):
    wid = lax.axis_index("s") * _NC + lax.axis_index("c")
    base = wid * _BPW
    pltpu.sync_copy(idx0_hbm.at[pl.ds(base, _BPW)], idx0_v)
    pltpu.sync_copy(idx1_hbm.at[pl.ds(base, _BPW)], idx1_v)
    pltpu.sync_copy(idx2_hbm.at[pl.ds(base, _BPW)], idx2_v)
    c0 = pltpu.async_copy(u0_hbm.at[idx0_v], r0_v, sem0)
    c1 = pltpu.async_copy(u1_hbm.at[idx1_v], r1_v, sem1)
    c2 = pltpu.async_copy(u2_hbm.at[idx2_v], r2_v, sem2)
    c0.wait()
    pltpu.sync_copy(r0_v, g0_hbm.at[pl.ds(base, _BPW)])
    c1.wait()
    pltpu.sync_copy(r1_v, g1_hbm.at[pl.ds(base, _BPW)])
    c2.wait()
    pltpu.sync_copy(r2_v, g2_hbm.at[pl.ds(base, _BPW)])


@functools.cache
def _sc_gather():
    return pl.kernel(
        _sc_gather_body,
        out_type=(
            jax.ShapeDtypeStruct((_B, _RP), jnp.float32),
            jax.ShapeDtypeStruct((_B, _RP), jnp.float32),
            jax.ShapeDtypeStruct((_B, _RP), jnp.float32),
        ),
        mesh=plsc.VectorSubcoreMesh(core_axis_name="c", subcore_axis_name="s"),
        scratch_types=[
            pltpu.VMEM((_BPW,), jnp.int32),
            pltpu.VMEM((_BPW,), jnp.int32),
            pltpu.VMEM((_BPW,), jnp.int32),
            pltpu.VMEM((_BPW, _RP), jnp.float32),
            pltpu.VMEM((_BPW, _RP), jnp.float32),
            pltpu.VMEM((_BPW, _RP), jnp.float32),
            pltpu.SemaphoreType.DMA,
            pltpu.SemaphoreType.DMA,
            pltpu.SemaphoreType.DMA,
        ],
    )


def _fast_cos(x):
    k = jnp.round(x * 0.6366197723675814)
    ki = k.astype(jnp.int32)
    r = x - k * 1.5707855224609375
    r = r - k * 1.0804334124e-5
    r = r - k * 6.0771e-11
    z = r * r
    cosp = ((2.443315711809948e-5 * z - 1.388731625493765e-3) * z
            + 4.166664568298827e-2) * z * z - 0.5 * z + 1.0
    sinp = (((-1.9515295891e-4 * z + 8.3321608736e-3) * z
             - 1.6666654611e-1) * z) * r + r
    m1 = ki & 1
    m2 = (ki >> 1) & 1
    res = jnp.where(m1 == 1, sinp, cosp)
    return jnp.where((m1 ^ m2) == 1, -res, res)


def _mlp_body(g0, g1, g2, t, w0, w1, w2, wt, bff, wout, bout, out):
    acc = jnp.dot(g0[:, :_R], w0[...], preferred_element_type=jnp.float32)
    acc += jnp.dot(g1[:, :_R], w1[...], preferred_element_type=jnp.float32)
    acc += jnp.dot(g2[:, :_R], w2[...], preferred_element_type=jnp.float32)
    acc += t[...] * wt[...]
    acc += bff[...]
    feat = _fast_cos(acc) * _SCALE
    out[...] = jnp.dot(feat, wout[...], preferred_element_type=jnp.float32) + bout[...]


def _mlp_call(blk):
    grid = _B // blk
    const = lambda shape: pl.BlockSpec(shape, lambda i: (0, 0))
    return pl.pallas_call(
        _mlp_body,
        grid=(grid,),
        in_specs=[
            pl.BlockSpec((blk, _RP), lambda i: (i, 0)),
            pl.BlockSpec((blk, _RP), lambda i: (i, 0)),
            pl.BlockSpec((blk, _RP), lambda i: (i, 0)),
            pl.BlockSpec((blk, 1), lambda i: (i, 0)),
            const((_R, _NFF)),
            const((_R, _NFF)),
            const((_R, _NFF)),
            const((1, _NFF)),
            const((1, _NFF)),
            const((_NFF, 1)),
            const((1, 1)),
        ],
        out_specs=pl.BlockSpec((blk, 1), lambda i: (i, 0)),
        out_shape=jax.ShapeDtypeStruct((_B, 1), jnp.float32),
    )


@jax.jit
def kernel(b_i_n, b_t_n, U0, U1, U2, W_ff, b_ff, W_out, b_out):
    idx0 = b_i_n[:, 0]
    idx1 = b_i_n[:, 1]
    idx2 = b_i_n[:, 2]
    tpad = ((0, 0), (0, _RP - _R))
    u0p = jnp.pad(U0, tpad)
    u1p = jnp.pad(U1, tpad)
    u2p = jnp.pad(U2, tpad)
    g0, g1, g2 = _sc_gather()(idx0, idx1, idx2, u0p, u1p, u2p)
    w0 = W_ff[0:_R]
    w1 = W_ff[_R:2 * _R]
    w2 = W_ff[2 * _R:3 * _R]
    wt = W_ff[3 * _R:3 * _R + 1]
    y = _mlp_call(1024)(
        g0, g1, g2, b_t_n.reshape(_B, 1),
        w0, w1, w2, wt, b_ff.reshape(1, _NFF),
        W_out, b_out.reshape(1, 1),
    )
    return y
